# baseline (speedup 1.0000x reference)
import jax
import jax.numpy as jnp
from jax import lax
from jax.experimental import pallas as pl
from jax.experimental.pallas import tpu as pltpu
from jax.experimental.pallas import tpu_sc as plsc

VOCAB = 1000
CHAR_DIM = 40
GAZET_DIM = 15
CTX = 21
SEQ = 16384
EMBED = CHAR_DIM + GAZET_DIM

NC, NS = 2, 16
NW = NC * NS
S_W = SEQ // NW
N_SL = S_W // 16


def _sc_body(tab_hbm, sent_hbm, gaz_hbm, out_hbm,
             tab_v, idx_v, gaz0, gaz1, out0, out1, sem_g, sem_o):
    wid = lax.axis_index("s") * NC + lax.axis_index("c")
    s0 = wid * S_W
    gazs = (gaz0, gaz1)
    outs = (out0, out1)

    pltpu.sync_copy(tab_hbm, tab_v)
    pltpu.sync_copy(sent_hbm.at[:, pl.ds(s0, S_W)], idx_v)
    pltpu.async_copy(gaz_hbm.at[0, :, pl.ds(s0, S_W)], gaz0, sem_g)

    def compute(c, gaz_v, out_v):
        jc = (c.astype(jnp.float32) + 1.0) * (1.0 / CTX)

        @plsc.parallel_loop(0, N_SL, unroll=4)
        def chunk(j):
            idx = idx_v[c, pl.ds(j * 16, 16)]
            for e in range(CHAR_DIM):
                ke = float(e + 1) / EMBED
                pe_ce = (1.0 - ke) - jc * (1.0 - 2.0 * ke)
                vals = plsc.load_gather(tab_v, [idx + e * VOCAB])
                out_v[e, pl.ds(j * 16, 16)] = vals + pe_ce

        @plsc.parallel_loop(0, N_SL, unroll=2)
        def chunk_g(j):
            for g in range(GAZET_DIM):
                e = CHAR_DIM + g
                ke = float(e + 1) / EMBED
                pe_ce = (1.0 - ke) - jc * (1.0 - 2.0 * ke)
                gv = gaz_v[g, pl.ds(j * 16, 16)]
                out_v[e, pl.ds(j * 16, 16)] = gv + pe_ce

    def pair(cc, carry):
        for b in range(2):
            c = cc * 2 + b

            @pl.when(c < CTX)
            def _():
                pltpu.make_async_copy(
                    gaz_hbm.at[0, :, pl.ds(s0, S_W)], gazs[b], sem_g
                ).wait()

                @pl.when(c + 1 < CTX)
                def _():
                    pltpu.async_copy(
                        gaz_hbm.at[c + 1, :, pl.ds(s0, S_W)], gazs[1 - b], sem_g
                    )

                @pl.when(c >= 2)
                def _():
                    pltpu.make_async_copy(
                        outs[b], out_hbm.at[0, :, pl.ds(s0, S_W)], sem_o
                    ).wait()

                compute(c, gazs[b], outs[b])
                pltpu.async_copy(
                    outs[b], out_hbm.at[c, :, pl.ds(s0, S_W)], sem_o
                )
        return carry

    lax.fori_loop(0, (CTX + 1) // 2, pair, 0)

    for b in range(2):
        pltpu.make_async_copy(
            outs[b], out_hbm.at[0, :, pl.ds(s0, S_W)], sem_o
        ).wait()


@jax.jit
def _run(tab_flat, sent_t, gaz_t):
    mesh = plsc.VectorSubcoreMesh(core_axis_name="c", subcore_axis_name="s")
    k = pl.kernel(
        _sc_body,
        out_type=jax.ShapeDtypeStruct((CTX, EMBED, SEQ), jnp.float32),
        mesh=mesh,
        scratch_types=[
            pltpu.VMEM((CHAR_DIM * VOCAB,), jnp.float32),
            pltpu.VMEM((CTX, S_W), jnp.int32),
            pltpu.VMEM((GAZET_DIM, S_W), jnp.float32),
            pltpu.VMEM((GAZET_DIM, S_W), jnp.float32),
            pltpu.VMEM((EMBED, S_W), jnp.float32),
            pltpu.VMEM((EMBED, S_W), jnp.float32),
            pltpu.SemaphoreType.DMA,
            pltpu.SemaphoreType.DMA,
        ],
        compiler_params=pltpu.CompilerParams(
            needs_layout_passes=False, use_tc_tiling_on_sc=False
        ),
    )
    return k(tab_flat, sent_t, gaz_t)


def kernel(sentence, gazet, char_table):
    tab_flat = char_table.T.reshape(CHAR_DIM * VOCAB)
    sent_t = jnp.swapaxes(sentence, 0, 1).astype(jnp.int32)
    gaz_t = jnp.transpose(gazet, (1, 2, 0))
    out = _run(tab_flat, sent_t, gaz_t)
    return jnp.transpose(out, (2, 0, 1))


# speedup vs baseline: 1.0121x; 1.0121x over previous
import jax
import jax.numpy as jnp
from jax import lax
from jax.experimental import pallas as pl
from jax.experimental.pallas import tpu as pltpu
from jax.experimental.pallas import tpu_sc as plsc

VOCAB = 1000
CHAR_DIM = 40
GAZET_DIM = 15
CTX = 21
SEQ = 16384
EMBED = CHAR_DIM + GAZET_DIM

NC, NS = 2, 16
NW = NC * NS
S_W = SEQ // NW
N_SL = S_W // 16


def _sc_body(tab_hbm, sent_hbm, gaz_hbm, out_hbm,
             tab_v, idx_v, gaz0, gaz1, out0, out1, sem_g, sem_o):
    wid = lax.axis_index("s") * NC + lax.axis_index("c")
    s0 = wid * S_W
    gazs = (gaz0, gaz1)
    outs = (out0, out1)

    pltpu.sync_copy(tab_hbm, tab_v)
    pltpu.sync_copy(sent_hbm.at[:, pl.ds(s0, S_W)], idx_v)
    pltpu.async_copy(gaz_hbm.at[0, :, pl.ds(s0, S_W)], gaz0, sem_g)

    def compute(c, gaz_v, out_v):
        jc = (c.astype(jnp.float32) + 1.0) * (1.0 / CTX)

        @plsc.parallel_loop(0, N_SL, unroll=2)
        def chunk(j):
            idx = idx_v[c, pl.ds(j * 16, 16)]
            for e in range(CHAR_DIM):
                ke = float(e + 1) / EMBED
                pe_ce = (1.0 - ke) - jc * (1.0 - 2.0 * ke)
                vals = plsc.load_gather(tab_v, [idx + e * VOCAB])
                out_v[e, pl.ds(j * 16, 16)] = vals + pe_ce

        @plsc.parallel_loop(0, N_SL, unroll=2)
        def chunk_g(j):
            for g in range(GAZET_DIM):
                e = CHAR_DIM + g
                ke = float(e + 1) / EMBED
                pe_ce = (1.0 - ke) - jc * (1.0 - 2.0 * ke)
                gv = gaz_v[g, pl.ds(j * 16, 16)]
                out_v[e, pl.ds(j * 16, 16)] = gv + pe_ce

    def pair(cc, carry):
        for b in range(2):
            c = cc * 2 + b

            @pl.when(c < CTX)
            def _():
                pltpu.make_async_copy(
                    gaz_hbm.at[0, :, pl.ds(s0, S_W)], gazs[b], sem_g
                ).wait()

                @pl.when(c + 1 < CTX)
                def _():
                    pltpu.async_copy(
                        gaz_hbm.at[c + 1, :, pl.ds(s0, S_W)], gazs[1 - b], sem_g
                    )

                @pl.when(c >= 2)
                def _():
                    pltpu.make_async_copy(
                        outs[b], out_hbm.at[0, :, pl.ds(s0, S_W)], sem_o
                    ).wait()

                compute(c, gazs[b], outs[b])
                pltpu.async_copy(
                    outs[b], out_hbm.at[c, :, pl.ds(s0, S_W)], sem_o
                )
        return carry

    lax.fori_loop(0, (CTX + 1) // 2, pair, 0)

    for b in range(2):
        pltpu.make_async_copy(
            outs[b], out_hbm.at[0, :, pl.ds(s0, S_W)], sem_o
        ).wait()


@jax.jit
def _run(tab_flat, sent_t, gaz_t):
    mesh = plsc.VectorSubcoreMesh(core_axis_name="c", subcore_axis_name="s")
    k = pl.kernel(
        _sc_body,
        out_type=jax.ShapeDtypeStruct((CTX, EMBED, SEQ), jnp.float32),
        mesh=mesh,
        scratch_types=[
            pltpu.VMEM((CHAR_DIM * VOCAB,), jnp.float32),
            pltpu.VMEM((CTX, S_W), jnp.int32),
            pltpu.VMEM((GAZET_DIM, S_W), jnp.float32),
            pltpu.VMEM((GAZET_DIM, S_W), jnp.float32),
            pltpu.VMEM((EMBED, S_W), jnp.float32),
            pltpu.VMEM((EMBED, S_W), jnp.float32),
            pltpu.SemaphoreType.DMA,
            pltpu.SemaphoreType.DMA,
        ],
        compiler_params=pltpu.CompilerParams(
            needs_layout_passes=False, use_tc_tiling_on_sc=False
        ),
    )
    return k(tab_flat, sent_t, gaz_t)


def kernel(sentence, gazet, char_table):
    tab_flat = char_table.T.reshape(CHAR_DIM * VOCAB)
    sent_t = jnp.swapaxes(sentence, 0, 1).astype(jnp.int32)
    gaz_t = jnp.transpose(gazet, (1, 2, 0))
    out = _run(tab_flat, sent_t, gaz_t)
    return jnp.transpose(out, (2, 0, 1))
